# baseline (speedup 1.0000x reference)
import jax
import jax.numpy as jnp
from jax.experimental import pallas as pl
from jax.experimental.pallas import tpu as pltpu

_CHUNK = 512
_DEPTH = 3


def _round_up(x, m):
    return ((x + m - 1) // m) * m


def _pad2d(arr, rows, cols):
    r, c = arr.shape
    if r == rows and c == cols:
        return arr
    return jnp.pad(arr, ((0, rows - r), (0, cols - c)))


def _make_body(nchunks):
    def body(a_ref, x_ref, w_ref, o_ref, abuf, obuf, xw_ref, asems, osems):
        for s in range(min(_DEPTH, nchunks)):
            pltpu.make_async_copy(
                a_ref.at[pl.ds(s * _CHUNK, _CHUNK), :],
                abuf.at[s], asems.at[s]).start()

        xb = x_ref[...].astype(jnp.bfloat16)
        wb = w_ref[...].astype(jnp.bfloat16)
        xw = jnp.dot(xb, wb, preferred_element_type=jnp.float32)
        xw_ref[...] = xw.astype(jnp.bfloat16)

        def loop(i, carry):
            slot = jax.lax.rem(i, _DEPTH)
            oslot = jax.lax.rem(i, 2)
            pltpu.make_async_copy(
                abuf.at[slot], abuf.at[slot], asems.at[slot]).wait()
            a = abuf[slot].astype(jnp.bfloat16)
            acc = jnp.dot(a, xw_ref[...], preferred_element_type=jnp.float32)

            @pl.when(i >= 2)
            def _():
                pltpu.make_async_copy(
                    obuf.at[oslot], obuf.at[oslot], osems.at[oslot]).wait()

            obuf[oslot] = jnp.maximum(acc, 0.0)
            pltpu.make_async_copy(
                obuf.at[oslot],
                o_ref.at[pl.ds(i * _CHUNK, _CHUNK), :],
                osems.at[oslot]).start()

            @pl.when(i + _DEPTH < nchunks)
            def _():
                pltpu.make_async_copy(
                    a_ref.at[pl.ds((i + _DEPTH) * _CHUNK, _CHUNK), :],
                    abuf.at[slot], asems.at[slot]).start()

            return carry

        jax.lax.fori_loop(0, nchunks, loop, 0)

        pltpu.make_async_copy(obuf.at[0], obuf.at[0], osems.at[0]).wait()
        pltpu.make_async_copy(obuf.at[1], obuf.at[1], osems.at[1]).wait()

    return body


@jax.jit
def kernel(a_hat, x, w):
    n = a_hat.shape[0]
    c_in = x.shape[1]
    c_out = w.shape[1]

    k_p = _round_up(n, 128)
    cin_p = _round_up(c_in, 128)
    cout_p = _round_up(c_out, 128)
    rows_p = _round_up(n, 2 * _CHUNK)
    nchunks = rows_p // _CHUNK

    a_p = _pad2d(a_hat, rows_p, k_p)
    x_p = _pad2d(x, k_p, cin_p)
    w_p = _pad2d(w, cin_p, cout_p)

    out_p = pl.pallas_call(
        _make_body(nchunks),
        out_shape=jax.ShapeDtypeStruct((rows_p, cout_p), jnp.float32),
        in_specs=[
            pl.BlockSpec(memory_space=pltpu.MemorySpace.HBM),
            pl.BlockSpec(memory_space=pltpu.MemorySpace.VMEM),
            pl.BlockSpec(memory_space=pltpu.MemorySpace.VMEM),
        ],
        out_specs=pl.BlockSpec(memory_space=pltpu.MemorySpace.HBM),
        scratch_shapes=[
            pltpu.VMEM((_DEPTH, _CHUNK, k_p), jnp.float32),
            pltpu.VMEM((2, _CHUNK, cout_p), jnp.float32),
            pltpu.VMEM((k_p, cout_p), jnp.bfloat16),
            pltpu.SemaphoreType.DMA((_DEPTH,)),
            pltpu.SemaphoreType.DMA((2,)),
        ],
    )(a_p, x_p, w_p)

    return out_p[:n, :c_out]


# speedup vs baseline: 1.0380x; 1.0380x over previous
import jax
import jax.numpy as jnp
from jax.experimental import pallas as pl
from jax.experimental.pallas import tpu as pltpu

_ROW_TILE = 512


def _round_up(x, m):
    return ((x + m - 1) // m) * m


def _pad2d(arr, rows, cols):
    r, c = arr.shape
    if r == rows and c == cols:
        return arr
    return jnp.pad(arr, ((0, rows - r), (0, cols - c)))


def _fused_gcn_kernel(a_ref, x_ref, w_ref, o_ref, xw_ref):
    @pl.when(pl.program_id(0) == 0)
    def _():
        xb = x_ref[...].astype(jnp.bfloat16)
        wb = w_ref[...].astype(jnp.bfloat16)
        xw = jnp.dot(xb, wb, preferred_element_type=jnp.float32)
        xw_ref[...] = xw.astype(jnp.bfloat16)

    a = a_ref[...].astype(jnp.bfloat16)
    acc = jnp.dot(a, xw_ref[...], preferred_element_type=jnp.float32)
    o_ref[...] = jnp.maximum(acc, 0.0)


@jax.jit
def kernel(a_hat, x, w):
    n = a_hat.shape[0]
    c_in = x.shape[1]
    c_out = w.shape[1]

    k_p = _round_up(n, 128)
    cin_p = _round_up(c_in, 128)
    cout_p = _round_up(c_out, 128)
    rows_p = _round_up(n, _ROW_TILE)
    nb = rows_p // _ROW_TILE

    a_p = _pad2d(a_hat, rows_p, k_p)
    x_p = _pad2d(x, k_p, cin_p)
    w_p = _pad2d(w, cin_p, cout_p)

    out_p = pl.pallas_call(
        _fused_gcn_kernel,
        out_shape=jax.ShapeDtypeStruct((rows_p, cout_p), jnp.float32),
        grid=(nb,),
        in_specs=[
            pl.BlockSpec((_ROW_TILE, k_p), lambda i: (i, 0)),
            pl.BlockSpec((k_p, cin_p), lambda i: (0, 0)),
            pl.BlockSpec((cin_p, cout_p), lambda i: (0, 0)),
        ],
        out_specs=pl.BlockSpec((_ROW_TILE, cout_p), lambda i: (i, 0)),
        scratch_shapes=[pltpu.VMEM((k_p, cout_p), jnp.bfloat16)],
        compiler_params=pltpu.CompilerParams(
            dimension_semantics=("arbitrary",)),
    )(a_p, x_p, w_p)

    return out_p[:n, :c_out]
